# baseline (device time: 15476 ns/iter reference)
import jax
import jax.numpy as jnp
from jax import lax
from jax.experimental import pallas as pl
from jax.experimental.pallas import tpu as pltpu

import os

D_ROWS = int(os.environ.get("AG_D", "320"))
N_FWD = int(os.environ.get("AG_NFWD", "8"))
N_PRIV = 2


def kernel(x):
    m, n = x.shape
    d = D_ROWS
    fwd = m - d
    fc = fwd // N_FWD
    priv = 2 * d - m
    pc = priv // N_PRIV
    assert fwd % N_FWD == 0 and priv % N_PRIV == 0
    NT = N_FWD + N_PRIV

    def body(x_ref, out_ref, p1_send, p1_recv, p2_send, p2_recv,
             x_sem, z_sem):
        my_x = lax.axis_index("x")
        my_y = lax.axis_index("y")
        my_z = lax.axis_index("z")
        nbr_y = (my_x, 1 - my_y, my_z)
        nbr_x = (1 - my_x, my_y, my_z)
        nbr_z = (my_x, my_y, 1 - my_z)
        hh = lax.rem(my_x + my_z, 2)

        barrier = pltpu.get_barrier_semaphore()
        pl.semaphore_signal(
            barrier, inc=1, device_id=nbr_y,
            device_id_type=pl.DeviceIdType.MESH,
        )
        pl.semaphore_signal(
            x_sem, inc=1, device_id=nbr_x,
            device_id_type=pl.DeviceIdType.MESH,
        )
        pl.semaphore_signal(
            z_sem, inc=1, device_id=nbr_z,
            device_id_type=pl.DeviceIdType.MESH,
        )
        pl.semaphore_wait(barrier, 1)

        my_off = my_y * m
        miss_off = (1 - my_y) * m

        fs_base = hh * d
        pv_base = fwd
        comp_base = (1 - hh) * d

        chunks = [(fs_base + c * fc, fc) for c in range(N_FWD)]
        chunks += [(pv_base + c * pc, pc) for c in range(N_PRIV)]

        sends1 = []
        for i, (row, nr) in enumerate(chunks):
            s = pltpu.make_async_remote_copy(
                src_ref=x_ref.at[pl.ds(row, nr)],
                dst_ref=out_ref.at[pl.ds(my_off + row, nr)],
                send_sem=p1_send.at[i],
                recv_sem=p1_recv.at[i],
                device_id=nbr_y,
                device_id_type=pl.DeviceIdType.MESH,
            )
            s.start()
            sends1.append(s)

        out_ref[pl.ds(my_off, m), :] = x_ref[:, :]

        pl.semaphore_wait(x_sem, 1)
        pl.semaphore_wait(z_sem, 1)

        sends2 = []
        for i, (row, nr) in enumerate(chunks):
            rows = pl.ds(miss_off + row, nr)
            recv = pltpu.make_async_remote_copy(
                src_ref=x_ref.at[pl.ds(0, nr)],
                dst_ref=out_ref.at[rows],
                send_sem=p1_send.at[i],
                recv_sem=p1_recv.at[i],
                device_id=nbr_y,
                device_id_type=pl.DeviceIdType.MESH,
            )
            recv.wait_recv()
            if i >= N_FWD:
                continue
            tgt = nbr_x if i % 2 == 0 else nbr_z
            s = pltpu.make_async_remote_copy(
                src_ref=out_ref.at[rows],
                dst_ref=out_ref.at[rows],
                send_sem=p2_send.at[i],
                recv_sem=p2_recv.at[i],
                device_id=tgt,
                device_id_type=pl.DeviceIdType.MESH,
            )
            s.start()
            sends2.append(s)

        for c in range(N_FWD):
            recv = pltpu.make_async_remote_copy(
                src_ref=x_ref.at[pl.ds(0, fc)],
                dst_ref=out_ref.at[pl.ds(miss_off + comp_base + c * fc, fc)],
                send_sem=p2_send.at[c],
                recv_sem=p2_recv.at[c],
                device_id=nbr_x if c % 2 == 0 else nbr_z,
                device_id_type=pl.DeviceIdType.MESH,
            )
            recv.wait_recv()

        for s in sends1 + sends2:
            s.wait_send()

    out_shape = jax.ShapeDtypeStruct((2 * m, n), x.dtype)
    return pl.pallas_call(
        body,
        out_shape=out_shape,
        in_specs=[pl.BlockSpec(memory_space=pltpu.VMEM)],
        out_specs=pl.BlockSpec(memory_space=pltpu.VMEM),
        scratch_shapes=[
            pltpu.SemaphoreType.DMA((NT,)),
            pltpu.SemaphoreType.DMA((NT,)),
            pltpu.SemaphoreType.DMA((N_FWD,)),
            pltpu.SemaphoreType.DMA((N_FWD,)),
            pltpu.SemaphoreType.REGULAR,
            pltpu.SemaphoreType.REGULAR,
        ],
        compiler_params=pltpu.CompilerParams(collective_id=0),
    )(x)


# device time: 14023 ns/iter; 1.1036x vs baseline; 1.1036x over previous
import jax
import jax.numpy as jnp
from jax import lax
from jax.experimental import pallas as pl
from jax.experimental.pallas import tpu as pltpu

import os

D_ROWS = int(os.environ.get("AG_D", "320"))
N_FWD = int(os.environ.get("AG_NFWD", "8"))
N_PRIV = 2


def kernel(x):
    m, n = x.shape
    d = D_ROWS
    fwd = m - d
    fc = fwd // N_FWD
    priv = 2 * d - m
    pc = priv // N_PRIV
    assert fwd % N_FWD == 0 and priv % N_PRIV == 0
    NT = N_FWD + N_PRIV

    def body(x_ref, out_ref, stage, p1_send, p1_recv, p2_send, p2_recv,
             cp_sems, x_sem, z_sem):
        my_x = lax.axis_index("x")
        my_y = lax.axis_index("y")
        my_z = lax.axis_index("z")
        nbr_y = (my_x, 1 - my_y, my_z)
        nbr_x = (1 - my_x, my_y, my_z)
        nbr_z = (my_x, my_y, 1 - my_z)
        hh = lax.rem(my_x + my_z, 2)

        barrier = pltpu.get_barrier_semaphore()
        pl.semaphore_signal(
            barrier, inc=1, device_id=nbr_y,
            device_id_type=pl.DeviceIdType.MESH,
        )
        pl.semaphore_signal(
            x_sem, inc=1, device_id=nbr_x,
            device_id_type=pl.DeviceIdType.MESH,
        )
        pl.semaphore_signal(
            z_sem, inc=1, device_id=nbr_z,
            device_id_type=pl.DeviceIdType.MESH,
        )
        pl.semaphore_wait(barrier, 1)

        my_off = my_y * m
        miss_off = (1 - my_y) * m

        fs_base = hh * d
        pv_base = fwd
        comp_base = (1 - hh) * d

        sends1 = []
        for i in range(N_FWD):
            row = fs_base + i * fc
            s = pltpu.make_async_remote_copy(
                src_ref=x_ref.at[pl.ds(row, fc)],
                dst_ref=stage.at[pl.ds(row, fc)],
                send_sem=p1_send.at[i],
                recv_sem=p1_recv.at[i],
                device_id=nbr_y,
                device_id_type=pl.DeviceIdType.MESH,
            )
            s.start()
            sends1.append(s)
        for i in range(N_PRIV):
            row = pv_base + i * pc
            s = pltpu.make_async_remote_copy(
                src_ref=x_ref.at[pl.ds(row, pc)],
                dst_ref=out_ref.at[pl.ds(my_off + row, pc)],
                send_sem=p1_send.at[N_FWD + i],
                recv_sem=p1_recv.at[N_FWD + i],
                device_id=nbr_y,
                device_id_type=pl.DeviceIdType.MESH,
            )
            s.start()
            sends1.append(s)

        own_cp = pltpu.make_async_copy(
            x_ref, out_ref.at[pl.ds(my_off, m)], cp_sems.at[N_FWD]
        )
        own_cp.start()

        pl.semaphore_wait(x_sem, 1)
        pl.semaphore_wait(z_sem, 1)

        sends2 = []
        copies = [own_cp]
        for i in range(N_FWD):
            row = fs_base + i * fc
            recv = pltpu.make_async_remote_copy(
                src_ref=x_ref.at[pl.ds(0, fc)],
                dst_ref=stage.at[pl.ds(row, fc)],
                send_sem=p1_send.at[i],
                recv_sem=p1_recv.at[i],
                device_id=nbr_y,
                device_id_type=pl.DeviceIdType.MESH,
            )
            recv.wait_recv()
            tgt = nbr_x if i % 2 == 0 else nbr_z
            s = pltpu.make_async_remote_copy(
                src_ref=stage.at[pl.ds(row, fc)],
                dst_ref=out_ref.at[pl.ds(miss_off + row, fc)],
                send_sem=p2_send.at[i],
                recv_sem=p2_recv.at[i],
                device_id=tgt,
                device_id_type=pl.DeviceIdType.MESH,
            )
            s.start()
            sends2.append(s)
            cp = pltpu.make_async_copy(
                stage.at[pl.ds(row, fc)],
                out_ref.at[pl.ds(miss_off + row, fc)],
                cp_sems.at[i],
            )
            cp.start()
            copies.append(cp)

        for i in range(N_PRIV):
            row = pv_base + i * pc
            recv = pltpu.make_async_remote_copy(
                src_ref=x_ref.at[pl.ds(0, pc)],
                dst_ref=out_ref.at[pl.ds(miss_off + row, pc)],
                send_sem=p1_send.at[N_FWD + i],
                recv_sem=p1_recv.at[N_FWD + i],
                device_id=nbr_y,
                device_id_type=pl.DeviceIdType.MESH,
            )
            recv.wait_recv()

        for i in range(N_FWD):
            recv = pltpu.make_async_remote_copy(
                src_ref=x_ref.at[pl.ds(0, fc)],
                dst_ref=out_ref.at[pl.ds(miss_off + comp_base + i * fc, fc)],
                send_sem=p2_send.at[i],
                recv_sem=p2_recv.at[i],
                device_id=nbr_x if i % 2 == 0 else nbr_z,
                device_id_type=pl.DeviceIdType.MESH,
            )
            recv.wait_recv()

        for s in sends1 + sends2:
            s.wait_send()
        for cp in copies:
            cp.wait()

    out_shape = jax.ShapeDtypeStruct((2 * m, n), x.dtype)
    return pl.pallas_call(
        body,
        out_shape=out_shape,
        in_specs=[pl.BlockSpec(memory_space=pltpu.VMEM)],
        out_specs=pl.BlockSpec(memory_space=pl.ANY),
        scratch_shapes=[
            pltpu.VMEM((m, n), x.dtype),
            pltpu.SemaphoreType.DMA((NT,)),
            pltpu.SemaphoreType.DMA((NT,)),
            pltpu.SemaphoreType.DMA((N_FWD,)),
            pltpu.SemaphoreType.DMA((N_FWD,)),
            pltpu.SemaphoreType.DMA((N_FWD + 1,)),
            pltpu.SemaphoreType.REGULAR,
            pltpu.SemaphoreType.REGULAR,
        ],
        compiler_params=pltpu.CompilerParams(collective_id=0),
    )(x)


# device time: 13845 ns/iter; 1.1178x vs baseline; 1.0129x over previous
import jax
import jax.numpy as jnp
from jax import lax
from jax.experimental import pallas as pl
from jax.experimental.pallas import tpu as pltpu

import os

D_ROWS = int(os.environ.get("AG_D", "320"))
N_FWD = int(os.environ.get("AG_NFWD", "8"))
N_PRIV = 2


def kernel(x):
    m, n = x.shape
    d = D_ROWS
    fwd = m - d
    fc = fwd // N_FWD
    priv = 2 * d - m
    pc = priv // N_PRIV
    assert fwd % N_FWD == 0 and priv % N_PRIV == 0
    NT = N_FWD + N_PRIV

    def body(x_ref, out_ref, p1_send, p1_recv, p2_send, p2_recv,
             x_sem, z_sem):
        my_x = lax.axis_index("x")
        my_y = lax.axis_index("y")
        my_z = lax.axis_index("z")
        nbr_y = (my_x, 1 - my_y, my_z)
        nbr_x = (1 - my_x, my_y, my_z)
        nbr_z = (my_x, my_y, 1 - my_z)
        hh = lax.rem(my_x + my_z, 2)

        barrier = pltpu.get_barrier_semaphore()
        pl.semaphore_signal(
            barrier, inc=1, device_id=nbr_y,
            device_id_type=pl.DeviceIdType.MESH,
        )
        pl.semaphore_signal(
            x_sem, inc=1, device_id=nbr_x,
            device_id_type=pl.DeviceIdType.MESH,
        )
        pl.semaphore_signal(
            z_sem, inc=1, device_id=nbr_z,
            device_id_type=pl.DeviceIdType.MESH,
        )
        pl.semaphore_wait(barrier, 1)

        my_off = my_y * m
        miss_off = (1 - my_y) * m

        fs_base = hh * d
        pv_base = fwd
        comp_base = (1 - hh) * d

        chunks = [(fs_base + c * fc, fc) for c in range(N_FWD)]
        chunks += [(pv_base + c * pc, pc) for c in range(N_PRIV)]

        sends1 = []
        for i, (row, nr) in enumerate(chunks):
            s = pltpu.make_async_remote_copy(
                src_ref=x_ref.at[pl.ds(row, nr)],
                dst_ref=out_ref.at[pl.ds(my_off + row, nr)],
                send_sem=p1_send.at[i],
                recv_sem=p1_recv.at[i],
                device_id=nbr_y,
                device_id_type=pl.DeviceIdType.MESH,
            )
            s.start()
            sends1.append(s)

        out_ref[pl.ds(my_off, m), :] = x_ref[:, :]

        pl.semaphore_wait(x_sem, 1)
        pl.semaphore_wait(z_sem, 1)

        sends2 = []
        for i, (row, nr) in enumerate(chunks):
            rows = pl.ds(miss_off + row, nr)
            recv = pltpu.make_async_remote_copy(
                src_ref=x_ref.at[pl.ds(0, nr)],
                dst_ref=out_ref.at[rows],
                send_sem=p1_send.at[i],
                recv_sem=p1_recv.at[i],
                device_id=nbr_y,
                device_id_type=pl.DeviceIdType.MESH,
            )
            recv.wait_recv()
            if i >= N_FWD:
                continue
            tgt = nbr_x if i % 2 == 0 else nbr_z
            s = pltpu.make_async_remote_copy(
                src_ref=out_ref.at[rows],
                dst_ref=out_ref.at[rows],
                send_sem=p2_send.at[i],
                recv_sem=p2_recv.at[i],
                device_id=tgt,
                device_id_type=pl.DeviceIdType.MESH,
            )
            s.start()
            sends2.append(s)

        for c in range(N_FWD):
            recv = pltpu.make_async_remote_copy(
                src_ref=x_ref.at[pl.ds(0, fc)],
                dst_ref=out_ref.at[pl.ds(miss_off + comp_base + c * fc, fc)],
                send_sem=p2_send.at[c],
                recv_sem=p2_recv.at[c],
                device_id=nbr_x if c % 2 == 0 else nbr_z,
                device_id_type=pl.DeviceIdType.MESH,
            )
            recv.wait_recv()

        for s in sends1 + sends2:
            s.wait_send()

    out_shape = jax.ShapeDtypeStruct((2 * m, n), x.dtype)
    return pl.pallas_call(
        body,
        out_shape=out_shape,
        in_specs=[pl.BlockSpec(memory_space=pltpu.VMEM)],
        out_specs=pl.BlockSpec(memory_space=pltpu.VMEM),
        scratch_shapes=[
            pltpu.SemaphoreType.DMA((NT,)),
            pltpu.SemaphoreType.DMA((NT,)),
            pltpu.SemaphoreType.DMA((N_FWD,)),
            pltpu.SemaphoreType.DMA((N_FWD,)),
            pltpu.SemaphoreType.REGULAR,
            pltpu.SemaphoreType.REGULAR,
        ],
        compiler_params=pltpu.CompilerParams(collective_id=0),
    )(x)
